# baseline (speedup 1.0000x reference)
import functools

import jax
import jax.numpy as jnp
from jax.experimental import pallas as pl
from jax.experimental.pallas import tpu as pltpu

_BM = 512
_BK = 512


def _hop1_body(L_ref, x_ref, t1_ref, acc_ref, *, nj, ni, last_rows, last_cols):
    i = pl.program_id(0)
    j = pl.program_id(1)

    @pl.when(j == 0)
    def _():
        acc_ref[...] = jnp.zeros_like(acc_ref)

    @pl.when(j < nj - 1)
    def _():
        acc_ref[...] += jnp.dot(L_ref[...].astype(jnp.bfloat16),
                                x_ref[...].astype(jnp.bfloat16),
                                preferred_element_type=jnp.float32)

    @pl.when(j == nj - 1)
    def _():
        acc = acc_ref[...] + jnp.dot(
            L_ref[:, :last_cols].astype(jnp.bfloat16),
            x_ref[:last_cols, :].astype(jnp.bfloat16),
            preferred_element_type=jnp.float32)
        rows = jax.lax.broadcasted_iota(jnp.int32, acc.shape, 0)
        acc = jnp.where((i == ni - 1) & (rows >= last_rows), 0.0, acc)
        t1_ref[...] = acc


def _hop2_body(L_ref, t1j_ref, t1i_ref, xi_ref, w0_ref, w1_ref, w2_ref,
               b_ref, o_ref, acc_ref, *, nj, last_cols):
    j = pl.program_id(1)

    @pl.when(j == 0)
    def _():
        acc_ref[...] = jnp.zeros_like(acc_ref)

    @pl.when(j < nj - 1)
    def _():
        acc_ref[...] += jnp.dot(L_ref[...].astype(jnp.bfloat16),
                                t1j_ref[...].astype(jnp.bfloat16),
                                preferred_element_type=jnp.float32)

    @pl.when(j == nj - 1)
    def _():
        xi = xi_ref[...]
        acc = acc_ref[...] + jnp.dot(
            L_ref[:, :last_cols].astype(jnp.bfloat16),
            t1j_ref[:last_cols, :].astype(jnp.bfloat16),
            preferred_element_type=jnp.float32)
        t2 = 2.0 * acc - xi
        o_ref[...] = (
            jnp.dot(xi, w0_ref[...], preferred_element_type=jnp.float32)
            + jnp.dot(t1i_ref[...], w1_ref[...],
                      preferred_element_type=jnp.float32)
            + jnp.dot(t2, w2_ref[...], preferred_element_type=jnp.float32)
            + b_ref[...])


def kernel(x, L_tilde, W0, W1, W2, b):
    n, din = x.shape
    dout = W0.shape[1]
    ni = pl.cdiv(n, _BM)
    nj = pl.cdiv(n, _BK)
    npad = ni * _BM
    last_rows = n - (ni - 1) * _BM
    last_cols = n - (nj - 1) * _BK

    xp = jnp.zeros((npad, din), x.dtype).at[:n].set(x)
    b2 = b.reshape(1, dout).astype(jnp.float32)

    seq = pltpu.CompilerParams(dimension_semantics=("arbitrary", "arbitrary"))

    t1 = pl.pallas_call(
        functools.partial(_hop1_body, nj=nj, ni=ni, last_rows=last_rows,
                          last_cols=last_cols),
        grid=(ni, nj),
        in_specs=[
            pl.BlockSpec((_BM, _BK), lambda i, j: (i, j)),
            pl.BlockSpec((_BK, din), lambda i, j: (j, 0)),
        ],
        out_specs=pl.BlockSpec((_BM, din), lambda i, j: (i, 0)),
        out_shape=jax.ShapeDtypeStruct((npad, din), jnp.float32),
        scratch_shapes=[pltpu.VMEM((_BM, din), jnp.float32)],
        compiler_params=seq,
    )(L_tilde, xp)

    out = pl.pallas_call(
        functools.partial(_hop2_body, nj=nj, last_cols=last_cols),
        grid=(ni, nj),
        in_specs=[
            pl.BlockSpec((_BM, _BK), lambda i, j: (i, j)),
            pl.BlockSpec((_BK, din), lambda i, j: (j, 0)),
            pl.BlockSpec((_BM, din), lambda i, j: (i, 0)),
            pl.BlockSpec((_BM, din), lambda i, j: (i, 0)),
            pl.BlockSpec((din, dout), lambda i, j: (0, 0)),
            pl.BlockSpec((din, dout), lambda i, j: (0, 0)),
            pl.BlockSpec((din, dout), lambda i, j: (0, 0)),
            pl.BlockSpec((1, dout), lambda i, j: (0, 0)),
        ],
        out_specs=pl.BlockSpec((_BM, dout), lambda i, j: (i, 0)),
        out_shape=jax.ShapeDtypeStruct((n, dout), jnp.float32),
        scratch_shapes=[pltpu.VMEM((_BM, dout), jnp.float32)],
        compiler_params=seq,
    )(L_tilde, t1, t1, xp, W0, W1, W2, b2)
    return out


# speedup vs baseline: 2.7898x; 2.7898x over previous
import functools

import jax
import jax.numpy as jnp
from jax.experimental import pallas as pl
from jax.experimental.pallas import tpu as pltpu

_BM = 256


def _hop1_body(L_ref, x_ref, t1_ref):
    t1_ref[...] = jnp.dot(L_ref[...].astype(jnp.bfloat16), x_ref[...],
                          preferred_element_type=jnp.float32)


def _hop2_body(L_ref, t1b_ref, t1i_ref, xi_ref, w0_ref, w1_ref, w2_ref,
               b_ref, o_ref):
    acc = jnp.dot(L_ref[...].astype(jnp.bfloat16), t1b_ref[...],
                  preferred_element_type=jnp.float32)
    xi = xi_ref[...]
    t2 = 2.0 * acc - xi
    o_ref[...] = (
        jnp.dot(xi, w0_ref[...], preferred_element_type=jnp.float32)
        + jnp.dot(t1i_ref[...], w1_ref[...], preferred_element_type=jnp.float32)
        + jnp.dot(t2, w2_ref[...], preferred_element_type=jnp.float32)
        + b_ref[...])


def kernel(x, L_tilde, W0, W1, W2, b):
    n, din = x.shape
    dout = W0.shape[1]
    ni = pl.cdiv(n, _BM)
    b2 = b.reshape(1, dout).astype(jnp.float32)
    xb = x.astype(jnp.bfloat16)

    seq = pltpu.CompilerParams(dimension_semantics=("arbitrary",))

    t1 = pl.pallas_call(
        _hop1_body,
        grid=(ni,),
        in_specs=[
            pl.BlockSpec((_BM, n), lambda i: (i, 0)),
            pl.BlockSpec((n, din), lambda i: (0, 0)),
        ],
        out_specs=pl.BlockSpec((_BM, din), lambda i: (i, 0)),
        out_shape=jax.ShapeDtypeStruct((n, din), jnp.float32),
        compiler_params=seq,
    )(L_tilde, xb)

    t1b = t1.astype(jnp.bfloat16)

    out = pl.pallas_call(
        _hop2_body,
        grid=(ni,),
        in_specs=[
            pl.BlockSpec((_BM, n), lambda i: (i, 0)),
            pl.BlockSpec((n, din), lambda i: (0, 0)),
            pl.BlockSpec((_BM, din), lambda i: (i, 0)),
            pl.BlockSpec((_BM, din), lambda i: (i, 0)),
            pl.BlockSpec((din, dout), lambda i: (0, 0)),
            pl.BlockSpec((din, dout), lambda i: (0, 0)),
            pl.BlockSpec((din, dout), lambda i: (0, 0)),
            pl.BlockSpec((1, dout), lambda i: (0, 0)),
        ],
        out_specs=pl.BlockSpec((_BM, dout), lambda i: (i, 0)),
        out_shape=jax.ShapeDtypeStruct((n, dout), jnp.float32),
        compiler_params=seq,
    )(L_tilde, t1b, t1, x, W0, W1, W2, b2)
    return out
